# Initial kernel scaffold; baseline (speedup 1.0000x reference)
import functools

import numpy as np
import jax
import jax.numpy as jnp
from jax import lax
from jax.experimental import pallas as pl
from jax.experimental.pallas import tpu as pltpu
from jax.experimental.pallas import tpu_sc as plsc

VOCAB = 100000
DIM = 128
S = 50
B = 1024
TOT = B * S

NC = 2
NS = 16
NW = NC * NS

PERW = TOT // NW
CHUNK = 80
NCHUNK = PERW // CHUNK

_rng = np.random.RandomState(0)
_mask_idx = _rng.choice(np.arange(B), int(B * 0.15))
_keep_np = np.ones((B, 1), dtype=np.float32)
_keep_np[_mask_idx] = 0.0


def _sc_gather_body(table_hbm, idx_hbm, out_hbm, idx_v, rows_v, gsem):
    wid = lax.axis_index("s") * NC + lax.axis_index("c")
    base_chunk = wid * NCHUNK
    base_row = wid * PERW
    pltpu.sync_copy(idx_hbm.at[pl.ds(base_chunk, NCHUNK)], idx_v)
    for c in range(NCHUNK):
        pltpu.async_copy(table_hbm.at[idx_v.at[c]], rows_v, gsem).wait()
        pltpu.sync_copy(rows_v, out_hbm.at[pl.ds(base_row + c * CHUNK, CHUNK)])


@jax.jit
def _sc_gather(tok_table, idx2d):
    mesh = plsc.VectorSubcoreMesh(core_axis_name="c", subcore_axis_name="s")
    return pl.kernel(
        _sc_gather_body,
        out_type=jax.ShapeDtypeStruct((TOT, DIM), jnp.float32),
        mesh=mesh,
        scratch_types=[
            pltpu.VMEM((NCHUNK, CHUNK), jnp.int32),
            pltpu.VMEM((CHUNK, DIM), jnp.float32),
            pltpu.SemaphoreType.DMA,
        ],
    )(tok_table, idx2d)


BBLK = 16


def _tc_ln_body(t_ref, pf_ref, pb_ref, kp_ref, g_ref, b_ref, o_ref):
    t = t_ref[...]
    pf = pf_ref[...]
    pb = pb_ref[...]
    kp = kp_ref[...]
    g = g_ref[...]
    bb = b_ref[...]

    def ln(v):
        u = jnp.mean(v, axis=-1, keepdims=True)
        s2 = jnp.mean((v - u) * (v - u), axis=-1, keepdims=True)
        return g[None] * ((v - u) * lax.rsqrt(s2 + 1e-12)) + bb[None]

    o_ref[:, 0] = ln(t + pf[None])
    o_ref[:, 1] = ln(t * kp[:, :, None])
    o_ref[:, 2] = ln(t + pb[None])


@jax.jit
def _tc_ln(x_tok, pos_f, pos_b, keep, gamma2d, beta2d):
    return pl.pallas_call(
        _tc_ln_body,
        grid=(B // BBLK,),
        in_specs=[
            pl.BlockSpec((BBLK, S, DIM), lambda i: (i, 0, 0)),
            pl.BlockSpec((S, DIM), lambda i: (0, 0)),
            pl.BlockSpec((S, DIM), lambda i: (0, 0)),
            pl.BlockSpec((BBLK, 1), lambda i: (i, 0)),
            pl.BlockSpec((1, DIM), lambda i: (0, 0)),
            pl.BlockSpec((1, DIM), lambda i: (0, 0)),
        ],
        out_specs=pl.BlockSpec((BBLK, 3, S, DIM), lambda i: (i, 0, 0, 0)),
        out_shape=jax.ShapeDtypeStruct((B, 3, S, DIM), jnp.float32),
    )(x_tok, pos_f, pos_b, keep, gamma2d, beta2d)


def kernel(x, tok_table, pos_table, gamma, beta):
    idx2d = x.astype(jnp.int32).reshape(NW * NCHUNK, CHUNK)
    x_tok = _sc_gather(tok_table, idx2d)
    pos_f = pos_table[:S]
    pos_b = pos_f[::-1]
    keep = jnp.asarray(_keep_np)
    out = _tc_ln(
        x_tok.reshape(B, S, DIM),
        pos_f,
        pos_b,
        keep,
        gamma.reshape(1, DIM),
        beta.reshape(1, DIM),
    )
    return out


# speedup vs baseline: 3.5089x; 3.5089x over previous
import functools

import numpy as np
import jax
import jax.numpy as jnp
from jax import lax
from jax.experimental import pallas as pl
from jax.experimental.pallas import tpu as pltpu
from jax.experimental.pallas import tpu_sc as plsc

VOCAB = 100000
DIM = 128
S = 50
B = 1024
TOT = B * S

NC = 2
NS = 16
NW = NC * NS

PERW = TOT // NW
CHUNK = 80
NCHUNK = PERW // CHUNK

_rng = np.random.RandomState(0)
_mask_idx = _rng.choice(np.arange(B), int(B * 0.15))
_keep_np = np.ones((B, 1), dtype=np.float32)
_keep_np[_mask_idx] = 0.0


def _sc_gather_body(table_hbm, idx_hbm, out_hbm, idx_v, rows_v, gsem):
    wid = lax.axis_index("s") * NC + lax.axis_index("c")
    base_row = wid * PERW
    pltpu.sync_copy(idx_hbm.at[pl.ds(base_row, PERW)], idx_v)
    for c in range(NCHUNK):
        pltpu.async_copy(
            table_hbm.at[idx_v.at[pl.ds(c * CHUNK, CHUNK)]], rows_v, gsem
        ).wait()
        pltpu.sync_copy(rows_v, out_hbm.at[pl.ds(base_row + c * CHUNK, CHUNK)])


@jax.jit
def _sc_gather(tok_table, idx_flat):
    mesh = plsc.VectorSubcoreMesh(core_axis_name="c", subcore_axis_name="s")
    return pl.kernel(
        _sc_gather_body,
        out_type=jax.ShapeDtypeStruct((TOT, DIM), jnp.float32),
        mesh=mesh,
        scratch_types=[
            pltpu.VMEM((PERW,), jnp.int32),
            pltpu.VMEM((CHUNK, DIM), jnp.float32),
            pltpu.SemaphoreType.DMA,
        ],
    )(tok_table, idx_flat)


BBLK = 16


def _tc_ln_body(t_ref, pf_ref, pb_ref, kp_ref, g_ref, b_ref, o_ref):
    t = t_ref[...]
    pf = pf_ref[...]
    pb = pb_ref[...]
    kp = kp_ref[...]
    g = g_ref[...]
    bb = b_ref[...]

    def ln(v):
        u = jnp.mean(v, axis=-1, keepdims=True)
        s2 = jnp.mean((v - u) * (v - u), axis=-1, keepdims=True)
        return g[None] * ((v - u) * lax.rsqrt(s2 + 1e-12)) + bb[None]

    o_ref[:, 0] = ln(t + pf[None])
    o_ref[:, 1] = ln(t * kp[:, :, None])
    o_ref[:, 2] = ln(t + pb[None])


@jax.jit
def _tc_ln(x_tok, pos_f, pos_b, keep, gamma2d, beta2d):
    return pl.pallas_call(
        _tc_ln_body,
        grid=(B // BBLK,),
        in_specs=[
            pl.BlockSpec((BBLK, S, DIM), lambda i: (i, 0, 0)),
            pl.BlockSpec((S, DIM), lambda i: (0, 0)),
            pl.BlockSpec((S, DIM), lambda i: (0, 0)),
            pl.BlockSpec((BBLK, 1), lambda i: (i, 0)),
            pl.BlockSpec((1, DIM), lambda i: (0, 0)),
            pl.BlockSpec((1, DIM), lambda i: (0, 0)),
        ],
        out_specs=pl.BlockSpec((BBLK, 3, S, DIM), lambda i: (i, 0, 0, 0)),
        out_shape=jax.ShapeDtypeStruct((B, 3, S, DIM), jnp.float32),
    )(x_tok, pos_f, pos_b, keep, gamma2d, beta2d)


def kernel(x, tok_table, pos_table, gamma, beta):
    idx_flat = x.astype(jnp.int32).reshape(TOT)
    x_tok = _sc_gather(tok_table, idx_flat)
    pos_f = pos_table[:S]
    pos_b = pos_f[::-1]
    keep = jnp.asarray(_keep_np)
    out = _tc_ln(
        x_tok.reshape(B, S, DIM),
        pos_f,
        pos_b,
        keep,
        gamma.reshape(1, DIM),
        beta.reshape(1, DIM),
    )
    return out


# speedup vs baseline: 3.9077x; 1.1136x over previous
import functools

import numpy as np
import jax
import jax.numpy as jnp
from jax import lax
from jax.experimental import pallas as pl
from jax.experimental.pallas import tpu as pltpu
from jax.experimental.pallas import tpu_sc as plsc

VOCAB = 100000
DIM = 128
S = 50
B = 1024
TOT = B * S

NC = 2
NS = 16
NW = NC * NS

PERW = TOT // NW
CHUNK = 80
NCHUNK = PERW // CHUNK

_rng = np.random.RandomState(0)
_mask_idx = _rng.choice(np.arange(B), int(B * 0.15))
_keep_np = np.ones((B, 1), dtype=np.float32)
_keep_np[_mask_idx] = 0.0


def _sc_gather_body(table_hbm, idx_hbm, out_hbm, idx_v, rows_v, gsem):
    wid = lax.axis_index("s") * NC + lax.axis_index("c")
    base_row = wid * PERW
    pltpu.sync_copy(idx_hbm.at[pl.ds(base_row, PERW)], idx_v)
    for c in range(NCHUNK):
        pltpu.async_copy(
            table_hbm.at[idx_v.at[pl.ds(c * CHUNK, CHUNK)]], rows_v, gsem
        ).wait()
        pltpu.sync_copy(rows_v, out_hbm.at[pl.ds(base_row + c * CHUNK, CHUNK)])


@jax.jit
def _sc_gather(tok_table, idx_flat):
    mesh = plsc.VectorSubcoreMesh(core_axis_name="c", subcore_axis_name="s")
    return pl.kernel(
        _sc_gather_body,
        out_type=jax.ShapeDtypeStruct((TOT, DIM), jnp.float32),
        mesh=mesh,
        scratch_types=[
            pltpu.VMEM((PERW,), jnp.int32),
            pltpu.VMEM((CHUNK, DIM), jnp.float32),
            pltpu.SemaphoreType.DMA,
        ],
    )(tok_table, idx_flat)


BBLK = 64


def _tc_ln_body(t_ref, pf_ref, pb_ref, kp_ref, g_ref, b_ref, o_ref):
    t = t_ref[...]
    pf = pf_ref[...]
    pb = pb_ref[...]
    kp = kp_ref[...]
    g = g_ref[...]
    bb = b_ref[...]

    def ln(v):
        u = jnp.mean(v, axis=-1, keepdims=True)
        s2 = jnp.mean((v - u) * (v - u), axis=-1, keepdims=True)
        return g[None] * ((v - u) * lax.rsqrt(s2 + 1e-12)) + bb[None]

    o_ref[:, 0] = ln(t + pf[None])
    o_ref[:, 1] = ln(t * kp[:, :, None])
    o_ref[:, 2] = ln(t + pb[None])


@jax.jit
def _tc_ln(x_tok, pos_f, pos_b, keep, gamma2d, beta2d):
    return pl.pallas_call(
        _tc_ln_body,
        grid=(B // BBLK,),
        in_specs=[
            pl.BlockSpec((BBLK, S, DIM), lambda i: (i, 0, 0)),
            pl.BlockSpec((S, DIM), lambda i: (0, 0)),
            pl.BlockSpec((S, DIM), lambda i: (0, 0)),
            pl.BlockSpec((BBLK, 1), lambda i: (i, 0)),
            pl.BlockSpec((1, DIM), lambda i: (0, 0)),
            pl.BlockSpec((1, DIM), lambda i: (0, 0)),
        ],
        out_specs=pl.BlockSpec((BBLK, 3, S, DIM), lambda i: (i, 0, 0, 0)),
        out_shape=jax.ShapeDtypeStruct((B, 3, S, DIM), jnp.float32),
    )(x_tok, pos_f, pos_b, keep, gamma2d, beta2d)


def kernel(x, tok_table, pos_table, gamma, beta):
    idx_flat = x.astype(jnp.int32).reshape(TOT)
    x_tok = _sc_gather(tok_table, idx_flat)
    pos_f = pos_table[:S]
    pos_b = pos_f[::-1]
    keep = jnp.asarray(_keep_np)
    out = _tc_ln(
        x_tok.reshape(B, S, DIM),
        pos_f,
        pos_b,
        keep,
        gamma.reshape(1, DIM),
        beta.reshape(1, DIM),
    )
    return out


# speedup vs baseline: 4.3433x; 1.1115x over previous
import functools

import numpy as np
import jax
import jax.numpy as jnp
from jax import lax
from jax.experimental import pallas as pl
from jax.experimental.pallas import tpu as pltpu
from jax.experimental.pallas import tpu_sc as plsc

VOCAB = 100000
DIM = 128
S = 50
B = 1024
TOT = B * S

NC = 2
NS = 16
NW = NC * NS

BPW = B // NW

_rng = np.random.RandomState(0)
_mask_idx = _rng.choice(np.arange(B), int(B * 0.15))
_keep_np = np.ones((B, 1), dtype=np.float32)
_keep_np[_mask_idx] = 0.0


def _sc_gather_body(table_hbm, idx_hbm, out_hbm, idx_v, rows_v, gsem):
    wid = lax.axis_index("s") * NC + lax.axis_index("c")
    base_b = wid * BPW
    pltpu.sync_copy(idx_hbm.at[pl.ds(base_b, BPW)], idx_v)

    def body(j, carry):
        pltpu.async_copy(table_hbm.at[idx_v.at[j]], rows_v, gsem).wait()
        pltpu.sync_copy(rows_v, out_hbm.at[base_b + j])
        return carry

    lax.fori_loop(0, BPW, body, 0)


@jax.jit
def _sc_gather(tok_table, idx):
    mesh = plsc.VectorSubcoreMesh(core_axis_name="c", subcore_axis_name="s")
    return pl.kernel(
        _sc_gather_body,
        out_type=jax.ShapeDtypeStruct((B, S, DIM), jnp.float32),
        mesh=mesh,
        scratch_types=[
            pltpu.VMEM((BPW, S), jnp.int32),
            pltpu.VMEM((S, DIM), jnp.float32),
            pltpu.SemaphoreType.DMA,
        ],
    )(tok_table, idx)


BBLK = 64


def _tc_ln_body(t_ref, pf_ref, pb_ref, kp_ref, g_ref, b_ref, o_ref):
    t = t_ref[...]
    pf = pf_ref[...]
    pb = pb_ref[...]
    kp = kp_ref[...]
    g = g_ref[...]
    bb = b_ref[...]

    def ln(v):
        u = jnp.mean(v, axis=-1, keepdims=True)
        s2 = jnp.mean((v - u) * (v - u), axis=-1, keepdims=True)
        return g[None] * ((v - u) * lax.rsqrt(s2 + 1e-12)) + bb[None]

    o_ref[:, 0] = ln(t + pf[None])
    o_ref[:, 1] = ln(t * kp[:, :, None])
    o_ref[:, 2] = ln(t + pb[None])


@jax.jit
def _tc_ln(x_tok, pos_f, pos_b, keep, gamma2d, beta2d):
    return pl.pallas_call(
        _tc_ln_body,
        grid=(B // BBLK,),
        in_specs=[
            pl.BlockSpec((BBLK, S, DIM), lambda i: (i, 0, 0)),
            pl.BlockSpec((S, DIM), lambda i: (0, 0)),
            pl.BlockSpec((S, DIM), lambda i: (0, 0)),
            pl.BlockSpec((BBLK, 1), lambda i: (i, 0)),
            pl.BlockSpec((1, DIM), lambda i: (0, 0)),
            pl.BlockSpec((1, DIM), lambda i: (0, 0)),
        ],
        out_specs=pl.BlockSpec((BBLK, 3, S, DIM), lambda i: (i, 0, 0, 0)),
        out_shape=jax.ShapeDtypeStruct((B, 3, S, DIM), jnp.float32),
    )(x_tok, pos_f, pos_b, keep, gamma2d, beta2d)


def kernel(x, tok_table, pos_table, gamma, beta):
    x_tok = _sc_gather(tok_table, x.astype(jnp.int32))
    pos_f = pos_table[:S]
    pos_b = pos_f[::-1]
    keep = jnp.asarray(_keep_np)
    out = _tc_ln(
        x_tok,
        pos_f,
        pos_b,
        keep,
        gamma.reshape(1, DIM),
        beta.reshape(1, DIM),
    )
    return out


# speedup vs baseline: 4.3464x; 1.0007x over previous
import functools

import numpy as np
import jax
import jax.numpy as jnp
from jax import lax
from jax.experimental import pallas as pl
from jax.experimental.pallas import tpu as pltpu
from jax.experimental.pallas import tpu_sc as plsc

VOCAB = 100000
DIM = 128
S = 50
B = 1024
TOT = B * S

NC = 2
NS = 16
NW = NC * NS

BPW = B // NW

_rng = np.random.RandomState(0)
_mask_idx = _rng.choice(np.arange(B), int(B * 0.15))
_keep_np = np.ones((B, 1), dtype=np.float32)
_keep_np[_mask_idx] = 0.0


def _sc_gather_body(table_hbm, idx_hbm, out_hbm, idx_v, rows_v, gsem):
    wid = lax.axis_index("s") * NC + lax.axis_index("c")
    base_b = wid * BPW
    pltpu.sync_copy(idx_hbm.at[pl.ds(base_b, BPW)], idx_v)

    def body(j, carry):
        pltpu.async_copy(table_hbm.at[idx_v.at[j]], rows_v, gsem).wait()
        pltpu.sync_copy(rows_v, out_hbm.at[base_b + j])
        return carry

    lax.fori_loop(0, BPW, body, 0)


@jax.jit
def _sc_gather(tok_table, idx):
    mesh = plsc.VectorSubcoreMesh(core_axis_name="c", subcore_axis_name="s")
    return pl.kernel(
        _sc_gather_body,
        out_type=jax.ShapeDtypeStruct((B, S, DIM), jnp.float32),
        mesh=mesh,
        scratch_types=[
            pltpu.VMEM((BPW, S), jnp.int32),
            pltpu.VMEM((S, DIM), jnp.float32),
            pltpu.SemaphoreType.DMA,
        ],
        compiler_params=pltpu.CompilerParams(use_tc_tiling_on_sc=True),
    )(tok_table, idx)


BBLK = 64


def _tc_ln_body(t_ref, pf_ref, pb_ref, kp_ref, g_ref, b_ref, o_ref):
    t = t_ref[...]
    pf = pf_ref[...]
    pb = pb_ref[...]
    kp = kp_ref[...]
    g = g_ref[...]
    bb = b_ref[...]

    def ln(v):
        u = jnp.mean(v, axis=-1, keepdims=True)
        s2 = jnp.mean((v - u) * (v - u), axis=-1, keepdims=True)
        return g[None] * ((v - u) * lax.rsqrt(s2 + 1e-12)) + bb[None]

    o_ref[:, 0] = ln(t + pf[None])
    o_ref[:, 1] = ln(t * kp[:, :, None])
    o_ref[:, 2] = ln(t + pb[None])


@jax.jit
def _tc_ln(x_tok, pos_f, pos_b, keep, gamma2d, beta2d):
    return pl.pallas_call(
        _tc_ln_body,
        grid=(B // BBLK,),
        in_specs=[
            pl.BlockSpec((BBLK, S, DIM), lambda i: (i, 0, 0)),
            pl.BlockSpec((S, DIM), lambda i: (0, 0)),
            pl.BlockSpec((S, DIM), lambda i: (0, 0)),
            pl.BlockSpec((BBLK, 1), lambda i: (i, 0)),
            pl.BlockSpec((1, DIM), lambda i: (0, 0)),
            pl.BlockSpec((1, DIM), lambda i: (0, 0)),
        ],
        out_specs=pl.BlockSpec((BBLK, 3, S, DIM), lambda i: (i, 0, 0, 0)),
        out_shape=jax.ShapeDtypeStruct((B, 3, S, DIM), jnp.float32),
    )(x_tok, pos_f, pos_b, keep, gamma2d, beta2d)


def kernel(x, tok_table, pos_table, gamma, beta):
    x_tok = _sc_gather(tok_table, x.astype(jnp.int32))
    pos_f = pos_table[:S]
    pos_b = pos_f[::-1]
    keep = jnp.asarray(_keep_np)
    out = _tc_ln(
        x_tok,
        pos_f,
        pos_b,
        keep,
        gamma.reshape(1, DIM),
        beta.reshape(1, DIM),
    )
    return out


# speedup vs baseline: 4.3476x; 1.0003x over previous
import functools

import numpy as np
import jax
import jax.numpy as jnp
from jax import lax
from jax.experimental import pallas as pl
from jax.experimental.pallas import tpu as pltpu
from jax.experimental.pallas import tpu_sc as plsc

VOCAB = 100000
DIM = 128
S = 50
B = 1024
TOT = B * S

NC = 2
NS = 16
NW = NC * NS

BPW = B // NW

_rng = np.random.RandomState(0)
_mask_idx = _rng.choice(np.arange(B), int(B * 0.15))
_keep_np = np.ones((B, 1), dtype=np.float32)
_keep_np[_mask_idx] = 0.0


def _sc_gather_body(table_hbm, idx_hbm, out_hbm, idx_v, rows_v, gsem):
    wid = lax.axis_index("s") * NC + lax.axis_index("c")
    base_b = wid * BPW
    pltpu.sync_copy(idx_hbm.at[pl.ds(base_b, BPW)], idx_v)

    def body(j, carry):
        pltpu.async_copy(table_hbm.at[idx_v.at[j]], rows_v, gsem).wait()
        pltpu.sync_copy(rows_v, out_hbm.at[base_b + j])
        return carry

    lax.fori_loop(0, BPW, body, 0)


def _sc_gather(tok_table, idx):
    mesh = plsc.VectorSubcoreMesh(core_axis_name="c", subcore_axis_name="s")
    return pl.kernel(
        _sc_gather_body,
        out_type=jax.ShapeDtypeStruct((B, S, DIM), jnp.float32),
        mesh=mesh,
        scratch_types=[
            pltpu.VMEM((BPW, S), jnp.int32),
            pltpu.VMEM((S, DIM), jnp.float32),
            pltpu.SemaphoreType.DMA,
        ],
        compiler_params=pltpu.CompilerParams(use_tc_tiling_on_sc=True),
    )(tok_table, idx)


BBLK = 64


def _tc_ln_body(t_ref, pf_ref, pb_ref, kp_ref, g_ref, b_ref, o_ref):
    t = t_ref[...]
    pf = pf_ref[...]
    pb = pb_ref[...]
    kp = kp_ref[...]
    g = g_ref[...]
    bb = b_ref[...]

    def ln(v):
        u = jnp.mean(v, axis=-1, keepdims=True)
        s2 = jnp.mean((v - u) * (v - u), axis=-1, keepdims=True)
        return g[None] * ((v - u) * lax.rsqrt(s2 + 1e-12)) + bb[None]

    o_ref[:, 0] = ln(t + pf[None])
    o_ref[:, 1] = ln(t * kp[:, :, None])
    o_ref[:, 2] = ln(t + pb[None])


def _tc_ln(x_tok, pos_f, pos_b, keep, gamma2d, beta2d):
    return pl.pallas_call(
        _tc_ln_body,
        grid=(B // BBLK,),
        in_specs=[
            pl.BlockSpec((BBLK, S, DIM), lambda i: (i, 0, 0)),
            pl.BlockSpec((S, DIM), lambda i: (0, 0)),
            pl.BlockSpec((S, DIM), lambda i: (0, 0)),
            pl.BlockSpec((BBLK, 1), lambda i: (i, 0)),
            pl.BlockSpec((1, DIM), lambda i: (0, 0)),
            pl.BlockSpec((1, DIM), lambda i: (0, 0)),
        ],
        out_specs=pl.BlockSpec((BBLK, 3, S, DIM), lambda i: (i, 0, 0, 0)),
        out_shape=jax.ShapeDtypeStruct((B, 3, S, DIM), jnp.float32),
    )(x_tok, pos_f, pos_b, keep, gamma2d, beta2d)


def kernel(x, tok_table, pos_table, gamma, beta):
    x_tok = _sc_gather(tok_table, x.astype(jnp.int32))
    pos_f = pos_table[:S]
    pos_b = pos_f[::-1]
    keep = jnp.asarray(_keep_np)
    out = _tc_ln(
        x_tok,
        pos_f,
        pos_b,
        keep,
        gamma.reshape(1, DIM),
        beta.reshape(1, DIM),
    )
    return out
